# baseline (speedup 1.0000x reference)
import functools

import jax
import jax.numpy as jnp
from jax import lax
from jax.experimental import pallas as pl
from jax.experimental.pallas import tpu as pltpu
from jax.experimental.pallas import tpu_sc as plsc

NUM_ENVS = 16384
NUM_JOINTS = 29
NUM_CORES = 2
NUM_SUBCORES = 16
NUM_WORKERS = NUM_CORES * NUM_SUBCORES
ROWS_PER_WORKER = NUM_ENVS // NUM_WORKERS


def _sc_body(inp_hbm, mem_hbm, def_hbm, out_hbm, mem_v, def_v):
    wid = lax.axis_index("s") * NUM_CORES + lax.axis_index("c")
    base = wid * ROWS_PER_WORKER

    pltpu.sync_copy(mem_hbm.at[pl.ds(base, ROWS_PER_WORKER)], mem_v)
    pltpu.sync_copy(def_hbm.at[pl.ds(base, ROWS_PER_WORKER)], def_v)

    @pl.when(wid == 0)
    def _():
        pltpu.sync_copy(inp_hbm, mem_v.at[0])

    def row(r, carry):
        a = mem_v[r, pl.ds(0, 16)] - def_v[r, pl.ds(0, 16)]
        mem_v[r, pl.ds(0, 16)] = a
        b = mem_v[r, pl.ds(13, 16)] - def_v[r, pl.ds(13, 16)]
        mem_v[r, pl.ds(13, 16)] = b
        return carry

    lax.fori_loop(0, ROWS_PER_WORKER, row, 0, unroll=4)

    pltpu.sync_copy(mem_v, out_hbm.at[pl.ds(base, ROWS_PER_WORKER)])


@jax.jit
def _sc_kernel(joint_pos_input, joint_pos, default_joint_pos):
    mesh = plsc.VectorSubcoreMesh(
        core_axis_name="c", subcore_axis_name="s",
        num_cores=NUM_CORES, num_subcores=NUM_SUBCORES)
    return pl.kernel(
        _sc_body,
        out_type=jax.ShapeDtypeStruct((NUM_ENVS, NUM_JOINTS), jnp.float32),
        mesh=mesh,
        scratch_types=[
            pltpu.VMEM((ROWS_PER_WORKER, NUM_JOINTS), jnp.float32),
            pltpu.VMEM((ROWS_PER_WORKER, NUM_JOINTS), jnp.float32),
        ],
    )(joint_pos_input, joint_pos, default_joint_pos)


def kernel(joint_pos_input, joint_pos, default_joint_pos):
    return _sc_kernel(joint_pos_input, joint_pos, default_joint_pos)


# speedup vs baseline: 1.7049x; 1.7049x over previous
import functools

import jax
import jax.numpy as jnp
from jax.experimental import pallas as pl

NUM_ENVS = 16384
NUM_JOINTS = 29
BLOCK_ROWS = 2048


def _body(inp_ref, jp_ref, djp_ref, out_ref):
    out_ref[...] = jp_ref[...] - djp_ref[...]

    @pl.when(pl.program_id(0) == 0)
    def _():
        out_ref[0:1, :] = inp_ref[...] - djp_ref[0:1, :]


@jax.jit
def _tc_kernel(joint_pos_input, joint_pos, default_joint_pos):
    inp2d = joint_pos_input.reshape(1, NUM_JOINTS)
    grid = (NUM_ENVS // BLOCK_ROWS,)
    return pl.pallas_call(
        _body,
        grid=grid,
        in_specs=[
            pl.BlockSpec((1, NUM_JOINTS), lambda i: (0, 0)),
            pl.BlockSpec((BLOCK_ROWS, NUM_JOINTS), lambda i: (i, 0)),
            pl.BlockSpec((BLOCK_ROWS, NUM_JOINTS), lambda i: (i, 0)),
        ],
        out_specs=pl.BlockSpec((BLOCK_ROWS, NUM_JOINTS), lambda i: (i, 0)),
        out_shape=jax.ShapeDtypeStruct((NUM_ENVS, NUM_JOINTS), jnp.float32),
    )(inp2d, joint_pos, default_joint_pos)


def kernel(joint_pos_input, joint_pos, default_joint_pos):
    return _tc_kernel(joint_pos_input, joint_pos, default_joint_pos)


# speedup vs baseline: 6.2264x; 3.6520x over previous
import jax
import jax.numpy as jnp
from jax.experimental import pallas as pl

NUM_ENVS = 16384
NUM_JOINTS = 29
BLOCK_COLS = 2048


def _body(inp_ref, jp_ref, djp_ref, out_ref):
    out_ref[...] = jp_ref[...] - djp_ref[...]

    @pl.when(pl.program_id(0) == 0)
    def _():
        out_ref[:, 0:1] = inp_ref[...] - djp_ref[:, 0:1]


@jax.jit
def _tc_kernel(joint_pos_input, joint_pos, default_joint_pos):
    jp_t = joint_pos.T
    djp_t = default_joint_pos.T
    inp_col = joint_pos_input.reshape(NUM_JOINTS, 1)
    grid = (NUM_ENVS // BLOCK_COLS,)
    out_t = pl.pallas_call(
        _body,
        grid=grid,
        in_specs=[
            pl.BlockSpec((NUM_JOINTS, 1), lambda i: (0, 0)),
            pl.BlockSpec((NUM_JOINTS, BLOCK_COLS), lambda i: (0, i)),
            pl.BlockSpec((NUM_JOINTS, BLOCK_COLS), lambda i: (0, i)),
        ],
        out_specs=pl.BlockSpec((NUM_JOINTS, BLOCK_COLS), lambda i: (0, i)),
        out_shape=jax.ShapeDtypeStruct((NUM_JOINTS, NUM_ENVS), jnp.float32),
    )(inp_col, jp_t, djp_t)
    return out_t.T


def kernel(joint_pos_input, joint_pos, default_joint_pos):
    return _tc_kernel(joint_pos_input, joint_pos, default_joint_pos)


# speedup vs baseline: 7.4324x; 1.1937x over previous
import jax
import jax.numpy as jnp
from jax.experimental import pallas as pl
from jax.experimental.pallas import tpu as pltpu

NUM_ENVS = 16384
NUM_JOINTS = 29
BLOCK_COLS = 2048


def _body(inp_ref, jp_ref, djp_ref, out_ref):
    out_ref[...] = jp_ref[...] - djp_ref[...]

    @pl.when(pl.program_id(0) == 0)
    def _():
        for j in range(NUM_JOINTS):
            out_ref[j : j + 1, 0:1] = inp_ref[j] - djp_ref[j : j + 1, 0:1]


@jax.jit
def _tc_kernel(joint_pos_input, joint_pos, default_joint_pos):
    jp_t = joint_pos.T
    djp_t = default_joint_pos.T
    grid = (NUM_ENVS // BLOCK_COLS,)
    out_t = pl.pallas_call(
        _body,
        grid=grid,
        in_specs=[
            pl.BlockSpec(memory_space=pltpu.SMEM),
            pl.BlockSpec((NUM_JOINTS, BLOCK_COLS), lambda i: (0, i)),
            pl.BlockSpec((NUM_JOINTS, BLOCK_COLS), lambda i: (0, i)),
        ],
        out_specs=pl.BlockSpec((NUM_JOINTS, BLOCK_COLS), lambda i: (0, i)),
        out_shape=jax.ShapeDtypeStruct((NUM_JOINTS, NUM_ENVS), jnp.float32),
    )(joint_pos_input, jp_t, djp_t)
    return out_t.T


def kernel(joint_pos_input, joint_pos, default_joint_pos):
    return _tc_kernel(joint_pos_input, joint_pos, default_joint_pos)
